# baseline (speedup 1.0000x reference)
import functools
import math

import jax
import jax.numpy as jnp
from jax import lax
from jax.experimental import pallas as pl
from jax.experimental.pallas import tpu as pltpu
from jax.experimental.pallas import tpu_sc as plsc

D_M = 64
SCALE = math.sqrt(D_M)
LANES = 16

try:
    _info = plsc.get_sparse_core_info()
    NC, NS = _info.num_cores, _info.num_subcores
except Exception:
    NC, NS = 2, 16
NW = NC * NS

MA, MB = 112, 88
_CP_A = tuple((s, s) for s in range(0, MA - LANES + 1, LANES))
_CP_B = tuple((s, MA + s) for s in (0, 16, 32, 48, 64, 72))
_G_A = tuple(s for s, _ in _CP_A)
_G_B = tuple(s for s, _ in _CP_B)


def _emb_body(S, rows_per_w,
              x_hbm, table_hbm, out_hbm,
              idx_v, idxp_v, h64_v, rows_v, outb_v, sem_in, sem_out):
    wid = lax.axis_index("s") * NC + lax.axis_index("c")
    row0 = wid * rows_per_w

    @pl.loop(0, rows_per_w // 8)
    def _blk(blk):
        r8 = row0 + blk * 8
        pltpu.sync_copy(x_hbm.at[pl.ds(r8, 8)], idx_v)

        @pl.loop(0, 4)
        def _rowpair(rp):
            for half in range(2):
                rr = 2 * rp + half
                for m2, cps in ((0, _CP_A), (1, _CP_B)):
                    m = 2 * half + m2
                    for dst, src in cps:
                        v = idx_v[(rr, pl.ds(src, LANES))]
                        sl = (m, 0, pl.ds(dst, LANES))
                        idxp_v[sl] = v >> 1
                        h64_v[sl] = (v & 1) << 6
            for m in range(4):
                n = MA if m % 2 == 0 else MB
                pltpu.make_async_copy(
                    table_hbm.at[idxp_v.at[m, 0, pl.ds(0, n)]],
                    rows_v.at[m, pl.ds(0, n)], sem_in.at[0]).start()
            for m in range(4):
                n = MA if m % 2 == 0 else MB
                pltpu.make_async_copy(
                    table_hbm.at[idxp_v.at[m, 0, pl.ds(0, n)]],
                    rows_v.at[m, pl.ds(0, n)], sem_in.at[0]).wait()

            lane = lax.iota(jnp.int32, LANES)
            for half in range(2):
                halfv = jnp.full((LANES,), half, jnp.int32)
                for m2, n, base, starts in (
                        (0, MA, 0, _G_A), (1, MB, MA, _G_B)):
                    m = 2 * half + m2
                    for gs in starts:
                        tvec = lane + gs
                        tokv = lane + (gs + base)
                        hvec = h64_v[(m, 0, pl.ds(gs, LANES))]

                        @pl.loop(0, D_M // 4, unroll=4)
                        def _dim(d4):
                            for dd in range(4):
                                d = d4 * 4 + dd
                                dv = jnp.full((LANES,), d, jnp.int32)
                                vals = plsc.load_gather(
                                    rows_v.at[m], [tvec, hvec + dv]) * SCALE
                                plsc.store_scatter(
                                    outb_v, [halfv, tokv, dv], vals)

            pltpu.make_async_copy(
                outb_v, out_hbm.at[pl.ds(r8 + 2 * rp, 2)],
                sem_out.at[0]).start()
            pltpu.make_async_copy(
                outb_v, out_hbm.at[pl.ds(r8 + 2 * rp, 2)],
                sem_out.at[0]).wait()


def _emb_lookup(x, table2):
    B0, S = x.shape
    rows_per_w = B0 // NW

    mesh = plsc.VectorSubcoreMesh(core_axis_name="c", subcore_axis_name="s")
    body = functools.partial(_emb_body, S, rows_per_w)
    return pl.kernel(
        body,
        out_type=jax.ShapeDtypeStruct((B0, S, D_M), jnp.float32),
        mesh=mesh,
        compiler_params=pltpu.CompilerParams(use_tc_tiling_on_sc=True,
                                             needs_layout_passes=False),
        scratch_types=[
            pltpu.VMEM((8, 200), jnp.int32),
            pltpu.VMEM((4, 1, MA), jnp.int32),
            pltpu.VMEM((4, 1, MA), jnp.int32),
            pltpu.VMEM((4, MA, 2 * D_M), jnp.float32),
            pltpu.VMEM((2, 200, D_M), jnp.float32),
            pltpu.SemaphoreType.DMA((1,)),
            pltpu.SemaphoreType.DMA((1,)),
        ],
    )(x, table2)


def kernel(x, table):
    t2 = table.reshape(table.shape[0] // 2, 2 * D_M)
    return _emb_lookup(x.astype(jnp.int32), t2)
